# Initial kernel scaffold; baseline (speedup 1.0000x reference)
import jax
import jax.numpy as jnp
from jax.experimental import pallas as pl


def kernel(card_indices, embedding_table):
    raise NotImplementedError("write your pallas kernel here")


# speedup vs baseline: 2.7622x; 2.7622x over previous
import functools

import jax
import jax.numpy as jnp
from jax import lax
from jax.experimental import pallas as pl
from jax.experimental.pallas import tpu as pltpu
from jax.experimental.pallas import tpu_sc as plsc

ROWS = 16384
COLS = 200
D = 32
B_TOTAL = ROWS * COLS
NUM_CORES = 2
NUM_SUBCORES = 16
NW = NUM_CORES * NUM_SUBCORES
B_PER_W = B_TOTAL // NW
CHUNK = 2048
N_CHUNKS = B_PER_W // CHUNK


def _make_gather():
    mesh = plsc.VectorSubcoreMesh(core_axis_name="c", subcore_axis_name="s")

    @functools.partial(
        pl.kernel,
        mesh=mesh,
        out_type=jax.ShapeDtypeStruct((B_TOTAL, D), jnp.float32),
        compiler_params=pltpu.CompilerParams(use_tc_tiling_on_sc=False),
        scratch_types=[
            pltpu.VMEM((CHUNK,), jnp.int32),
            pltpu.VMEM((CHUNK, D), jnp.float32),
            pltpu.SemaphoreType.DMA,
        ],
    )
    def gather_kernel(idx_hbm, table_hbm, out_hbm, idx_v, rows_v, sem):
        wid = lax.axis_index("s") * NUM_CORES + lax.axis_index("c")
        base0 = wid * B_PER_W

        def body(i, carry):
            base = base0 + i * CHUNK
            pltpu.sync_copy(idx_hbm.at[pl.ds(base, CHUNK)], idx_v)
            pltpu.async_copy(table_hbm.at[idx_v], rows_v, sem).wait()
            pltpu.sync_copy(rows_v, out_hbm.at[pl.ds(base, CHUNK)])
            return carry

        lax.fori_loop(0, N_CHUNKS, body, 0)

    return gather_kernel


_gather = _make_gather()


@jax.jit
def kernel(card_indices, embedding_table):
    flat_idx = card_indices.reshape(B_TOTAL).astype(jnp.int32)
    out = _gather(flat_idx, embedding_table)
    return out.reshape(ROWS, COLS, D)


# speedup vs baseline: 6.9168x; 2.5041x over previous
import functools

import jax
import jax.numpy as jnp
from jax import lax
from jax.experimental import pallas as pl
from jax.experimental.pallas import tpu as pltpu
from jax.experimental.pallas import tpu_sc as plsc

ROWS = 16384
COLS = 200
D = 32
VOCAB_ROWS = 53
B_TOTAL = ROWS * COLS
NUM_CORES = 2
NUM_SUBCORES = 16
NW = NUM_CORES * NUM_SUBCORES
B_PER_W = B_TOTAL // NW
CHUNK = 1600
N_PAIRS = B_PER_W // (2 * CHUNK)


def _make_gather():
    mesh = plsc.VectorSubcoreMesh(core_axis_name="c", subcore_axis_name="s")

    @functools.partial(
        pl.kernel,
        mesh=mesh,
        out_type=jax.ShapeDtypeStruct((B_TOTAL, D), jnp.float32),
        compiler_params=pltpu.CompilerParams(use_tc_tiling_on_sc=False),
        scratch_types=[
            pltpu.VMEM_SHARED((VOCAB_ROWS, D), jnp.float32),
            pltpu.VMEM((CHUNK,), jnp.int32),
            pltpu.VMEM((CHUNK,), jnp.int32),
            pltpu.VMEM((CHUNK, D), jnp.float32),
            pltpu.VMEM((CHUNK, D), jnp.float32),
            pltpu.SemaphoreType.DMA,
            pltpu.SemaphoreType.DMA,
            pltpu.SemaphoreType.DMA,
        ],
    )
    def gather_kernel(idx_hbm, table_hbm, out_hbm,
                      table_s, idx0_v, idx1_v, rows0_v, rows1_v,
                      sem_g, sem_o0, sem_o1):
        sid = lax.axis_index("s")
        cid = lax.axis_index("c")
        wid = sid * NUM_CORES + cid
        base0 = wid * B_PER_W

        @pl.when(sid == 0)
        def _():
            pltpu.sync_copy(table_hbm, table_s)

        plsc.subcore_barrier()

        def body(j, carry):
            ba = base0 + (2 * j) * CHUNK
            bb = ba + CHUNK

            @pl.when(j > 0)
            def _():
                pltpu.make_async_copy(
                    out_hbm.at[pl.ds(ba, CHUNK)], rows0_v, sem_o0).wait()
            pltpu.sync_copy(idx_hbm.at[pl.ds(ba, CHUNK)], idx0_v)
            pltpu.async_copy(table_s.at[idx0_v], rows0_v, sem_g).wait()
            pltpu.async_copy(rows0_v, out_hbm.at[pl.ds(ba, CHUNK)], sem_o0)

            @pl.when(j > 0)
            def _():
                pltpu.make_async_copy(
                    out_hbm.at[pl.ds(bb, CHUNK)], rows1_v, sem_o1).wait()
            pltpu.sync_copy(idx_hbm.at[pl.ds(bb, CHUNK)], idx1_v)
            pltpu.async_copy(table_s.at[idx1_v], rows1_v, sem_g).wait()
            pltpu.async_copy(rows1_v, out_hbm.at[pl.ds(bb, CHUNK)], sem_o1)
            return carry

        lax.fori_loop(0, N_PAIRS, body, 0)

        pltpu.make_async_copy(out_hbm.at[pl.ds(base0, CHUNK)], rows0_v,
                              sem_o0).wait()
        pltpu.make_async_copy(out_hbm.at[pl.ds(base0, CHUNK)], rows1_v,
                              sem_o1).wait()

    return gather_kernel


_gather = _make_gather()


@jax.jit
def kernel(card_indices, embedding_table):
    flat_idx = card_indices.reshape(B_TOTAL).astype(jnp.int32)
    out = _gather(flat_idx, embedding_table)
    return out.reshape(ROWS, COLS, D)
